# Initial kernel scaffold; baseline (speedup 1.0000x reference)
import jax
import jax.numpy as jnp
from jax.experimental import pallas as pl


def kernel(x, edge_index, W1, b1, W2, b2, W3, b3):
    raise NotImplementedError("write your pallas kernel here")


# speedup vs baseline: 38.7692x; 38.7692x over previous
import functools

import jax
import jax.numpy as jnp
from jax import lax
from jax.experimental import pallas as pl
from jax.experimental.pallas import tpu as pltpu
from jax.experimental.pallas import tpu_sc as plsc

N = 10000
N_PAD = 10240
E = 320000
CHUNK = 128
J_ROWS = E // CHUNK
NBUF = 2
TILES = 16
RPT = N_PAD // TILES

_mesh = plsc.VectorSubcoreMesh(core_axis_name="c", subcore_axis_name="s")
_sc_params = pltpu.CompilerParams(use_tc_tiling_on_sc=False)


def _row_span(wid):
    base = 78 * wid + jnp.minimum(wid, 4)
    jw = jnp.where(wid < 4, 79, 78)
    return base, jw


def _load_rows(hbm, vmem, base, jw):
    pltpu.sync_copy(hbm.at[pl.ds(base, 78)], vmem.at[pl.ds(0, 78)])

    @pl.when(jw > 78)
    def _():
        pltpu.sync_copy(hbm.at[pl.ds(base + 78, 1)], vmem.at[pl.ds(78, 1)])


@functools.partial(
    pl.kernel,
    mesh=_mesh,
    compiler_params=_sc_params,
    out_type=jax.ShapeDtypeStruct((2, N_PAD), jnp.float32),
    scratch_types=[
        pltpu.VMEM((80, CHUNK), jnp.int32),
        pltpu.VMEM((CHUNK,), jnp.float32),
        pltpu.VMEM((RPT,), jnp.float32),
        pltpu.VMEM_SHARED((N_PAD,), jnp.float32),
    ],
)
def _deg_sc(dst_hbm, out_hbm, dst_v, ones_v, zer_v, acc):
    c = lax.axis_index("c")
    s = lax.axis_index("s")
    wid = c * TILES + s
    base, jw = _row_span(wid)
    for k in range(RPT // 16):
        zer_v[pl.ds(k * 16, 16)] = jnp.zeros((16,), jnp.float32)
    for k in range(CHUNK // 16):
        ones_v[pl.ds(k * 16, 16)] = jnp.ones((16,), jnp.float32)
    pltpu.sync_copy(zer_v, acc.at[pl.ds(s * RPT, RPT)])
    _load_rows(dst_hbm, dst_v, base, jw)
    plsc.subcore_barrier()

    def body(j, carry):
        pltpu.sync_copy(ones_v, acc.at[dst_v.at[j]], add=True)
        return carry

    lax.fori_loop(0, jw, body, 0)
    plsc.subcore_barrier()
    pltpu.sync_copy(acc.at[pl.ds(s * RPT, RPT)], out_hbm.at[c, pl.ds(s * RPT, RPT)])


def _make_agg(D):
    @functools.partial(
        pl.kernel,
        mesh=_mesh,
        compiler_params=_sc_params,
        out_type=jax.ShapeDtypeStruct((2, N_PAD, D), jnp.float32),
        scratch_types=[
            pltpu.VMEM((80, CHUNK), jnp.int32),
            pltpu.VMEM((80, CHUNK), jnp.int32),
            [pltpu.VMEM((CHUNK, D), jnp.float32)] * NBUF,
            pltpu.VMEM((32, D), jnp.float32),
            pltpu.VMEM_SHARED((N_PAD, D), jnp.float32),
            pltpu.VMEM_SHARED((N_PAD, D), jnp.float32),
            [pltpu.SemaphoreType.DMA] * NBUF,
            [pltpu.SemaphoreType.DMA] * NBUF,
        ],
    )
    def agg(y_hbm, src_hbm, dst_hbm, out_hbm,
            src_v, dst_v, bufs, zer_v, acc, y_s, gs, ss):
        c = lax.axis_index("c")
        s = lax.axis_index("s")
        wid = c * TILES + s
        base, jw = _row_span(wid)
        for r in range(32):
            for k in range(D // 16):
                zer_v[r, pl.ds(k * 16, 16)] = jnp.zeros((16,), jnp.float32)
        for i in range(RPT // 32):
            pltpu.sync_copy(zer_v, acc.at[pl.ds(s * RPT + i * 32, 32)])
        pltpu.sync_copy(y_hbm.at[pl.ds(s * RPT, RPT)], y_s.at[pl.ds(s * RPT, RPT)])
        _load_rows(src_hbm, src_v, base, jw)
        _load_rows(dst_hbm, dst_v, base, jw)
        plsc.subcore_barrier()

        def fire_gather(j, b):
            pltpu.async_copy(y_s.at[src_v.at[j]], bufs[b], gs[b])

        def drain(b, sem):
            pltpu.make_async_copy(y_hbm.at[pl.ds(0, CHUNK)], bufs[b], sem).wait()

        for b in range(NBUF):
            fire_gather(b, b)

        def body(i, carry):
            j0 = NBUF * i
            for b in range(NBUF):
                j = j0 + b

                @pl.when(j < jw)
                def _():
                    drain(b, gs[b])
                    pltpu.async_copy(bufs[b], acc.at[dst_v.at[j]], ss[b], add=True)

            for b in range(NBUF):
                j = j0 + b

                @pl.when(j < jw)
                def _():
                    drain(b, ss[b])

                    @pl.when(j + NBUF < jw)
                    def _():
                        fire_gather(j + NBUF, b)

            return carry

        lax.fori_loop(0, 80 // NBUF, body, 0)
        plsc.subcore_barrier()
        for i in range(RPT // 128):
            r0 = s * RPT + i * 128
            pltpu.sync_copy(acc.at[pl.ds(r0, 128)], out_hbm.at[c, pl.ds(r0, 128)])

    return agg


_agg64 = _make_agg(64)
_agg16 = _make_agg(16)


def _dis_of(deg_ref):
    return lax.rsqrt(deg_ref[0, :N] + deg_ref[1, :N] + 1.0)[:, None]


def _b1_body(x_ref, deg_ref, w_ref, y_ref):
    xw = jnp.dot(x_ref[...], w_ref[...], preferred_element_type=jnp.float32)
    y_ref[pl.ds(0, N), :] = xw * _dis_of(deg_ref)
    y_ref[pl.ds(N, N_PAD - N), :] = jnp.zeros((N_PAD - N, y_ref.shape[1]),
                                              jnp.float32)


def _mid_body(agg_ref, y_ref, deg_ref, b_ref, w_ref, out_ref):
    dis = _dis_of(deg_ref)
    a = agg_ref[0, :N, :] + agg_ref[1, :N, :] + y_ref[:N, :]
    h = jnp.maximum(a * dis + b_ref[...][None, :], 0.0)
    out_ref[pl.ds(0, N), :] = jnp.dot(
        h, w_ref[...], preferred_element_type=jnp.float32) * dis
    out_ref[pl.ds(N, N_PAD - N), :] = jnp.zeros((N_PAD - N, out_ref.shape[1]),
                                                jnp.float32)


def _b4_body(agg_ref, y_ref, deg_ref, b_ref, out_ref):
    dis = _dis_of(deg_ref)
    t = (agg_ref[0, :N, :2] + agg_ref[1, :N, :2] + y_ref[:N, :2]) * dis
    t = t + b_ref[...][None, :]
    m = jnp.max(t, axis=1, keepdims=True)
    lse = m + jnp.log(jnp.sum(jnp.exp(t - m), axis=1, keepdims=True))
    out_ref[...] = t - lse


def _tc(body, out_rows, out_cols):
    return pl.pallas_call(
        body, out_shape=jax.ShapeDtypeStruct((out_rows, out_cols), jnp.float32))


def kernel(x, edge_index, W1, b1, W2, b2, W3, b3):
    ei = edge_index.astype(jnp.int32)
    src2 = ei[0].reshape(J_ROWS, CHUNK)
    dst2 = ei[1].reshape(J_ROWS, CHUNK)
    W3p = jnp.pad(W3, ((0, 0), (0, 16 - W3.shape[1])))

    deg = _deg_sc(dst2)
    y1 = _tc(_b1_body, N_PAD, 64)(x, deg, W1)
    a1 = _agg64(y1, src2, dst2)
    y2 = _tc(_mid_body, N_PAD, 64)(a1, y1, deg, b1, W2)
    a2 = _agg64(y2, src2, dst2)
    y3 = _tc(_mid_body, N_PAD, 16)(a2, y2, deg, b2, W3p)
    a3 = _agg16(y3, src2, dst2)
    out = _tc(_b4_body, N, 2)(a3, y3, deg, b3)
    return out


# speedup vs baseline: 39.0473x; 1.0072x over previous
import functools

import jax
import jax.numpy as jnp
from jax import lax
from jax.experimental import pallas as pl
from jax.experimental.pallas import tpu as pltpu
from jax.experimental.pallas import tpu_sc as plsc

N = 10000
N_PAD = 10240
E = 320000
CHUNK = 128
J_ROWS = E // CHUNK
NBUF = 2
TILES = 16
RPT = N_PAD // TILES

_mesh = plsc.VectorSubcoreMesh(core_axis_name="c", subcore_axis_name="s")
_sc_params = pltpu.CompilerParams(use_tc_tiling_on_sc=False)


def _row_span(wid):
    base = 78 * wid + jnp.minimum(wid, 4)
    jw = jnp.where(wid < 4, 79, 78)
    return base, jw


def _load_rows(hbm, vmem, base, jw):
    pltpu.sync_copy(hbm.at[pl.ds(base, 78)], vmem.at[pl.ds(0, 78)])

    @pl.when(jw > 78)
    def _():
        pltpu.sync_copy(hbm.at[pl.ds(base + 78, 1)], vmem.at[pl.ds(78, 1)])


@functools.partial(
    pl.kernel,
    mesh=_mesh,
    compiler_params=_sc_params,
    out_type=jax.ShapeDtypeStruct((2, N_PAD), jnp.float32),
    scratch_types=[
        pltpu.VMEM((80, CHUNK), jnp.int32),
        pltpu.VMEM((CHUNK,), jnp.float32),
        pltpu.VMEM((RPT,), jnp.float32),
        pltpu.VMEM_SHARED((N_PAD,), jnp.float32),
    ],
)
def _deg_sc(dst_hbm, out_hbm, dst_v, ones_v, zer_v, acc):
    c = lax.axis_index("c")
    s = lax.axis_index("s")
    wid = c * TILES + s
    base, jw = _row_span(wid)
    for k in range(RPT // 16):
        zer_v[pl.ds(k * 16, 16)] = jnp.zeros((16,), jnp.float32)
    for k in range(CHUNK // 16):
        ones_v[pl.ds(k * 16, 16)] = jnp.ones((16,), jnp.float32)
    pltpu.sync_copy(zer_v, acc.at[pl.ds(s * RPT, RPT)])
    _load_rows(dst_hbm, dst_v, base, jw)
    plsc.subcore_barrier()

    def body(j, carry):
        pltpu.sync_copy(ones_v, acc.at[dst_v.at[j]], add=True)
        return carry

    lax.fori_loop(0, jw, body, 0)
    plsc.subcore_barrier()
    pltpu.sync_copy(acc.at[pl.ds(s * RPT, RPT)], out_hbm.at[c, pl.ds(s * RPT, RPT)])


def _make_agg(D):
    @functools.partial(
        pl.kernel,
        mesh=_mesh,
        compiler_params=_sc_params,
        out_type=jax.ShapeDtypeStruct((2, N_PAD, D), jnp.float32),
        scratch_types=[
            pltpu.VMEM((80, CHUNK), jnp.int32),
            pltpu.VMEM((80, CHUNK), jnp.int32),
            [pltpu.VMEM((CHUNK, D), jnp.float32)] * NBUF,
            pltpu.VMEM((128, D), jnp.float32),
            pltpu.VMEM_SHARED((N_PAD, D), jnp.float32),
            pltpu.VMEM_SHARED((N_PAD, D), jnp.float32),
            [pltpu.SemaphoreType.DMA] * NBUF,
            [pltpu.SemaphoreType.DMA] * NBUF,
        ],
    )
    def agg(y_hbm, src_hbm, dst_hbm, out_hbm,
            src_v, dst_v, bufs, zer_v, acc, y_s, gs, ss):
        c = lax.axis_index("c")
        s = lax.axis_index("s")
        wid = c * TILES + s
        base, jw = _row_span(wid)
        for r in range(128):
            for k in range(D // 16):
                zer_v[r, pl.ds(k * 16, 16)] = jnp.zeros((16,), jnp.float32)
        for i in range(RPT // 128):
            pltpu.sync_copy(zer_v, acc.at[pl.ds(s * RPT + i * 128, 128)])
        pltpu.sync_copy(y_hbm.at[pl.ds(s * RPT, RPT)], y_s.at[pl.ds(s * RPT, RPT)])
        _load_rows(src_hbm, src_v, base, jw)
        _load_rows(dst_hbm, dst_v, base, jw)
        plsc.subcore_barrier()

        def fire_gather(j, b):
            pltpu.async_copy(y_s.at[src_v.at[j]], bufs[b], gs[b])

        def drain(b, sem):
            pltpu.make_async_copy(y_hbm.at[pl.ds(0, CHUNK)], bufs[b], sem).wait()

        for b in range(NBUF):
            fire_gather(b, b)

        def body(i, carry):
            j0 = NBUF * i
            for b in range(NBUF):
                j = j0 + b

                @pl.when(j < jw)
                def _():
                    drain(b, gs[b])
                    pltpu.async_copy(bufs[b], acc.at[dst_v.at[j]], ss[b], add=True)

            for b in range(NBUF):
                j = j0 + b

                @pl.when(j < jw)
                def _():
                    drain(b, ss[b])

                    @pl.when(j + NBUF < jw)
                    def _():
                        fire_gather(j + NBUF, b)

            return carry

        lax.fori_loop(0, 80 // NBUF, body, 0)
        plsc.subcore_barrier()
        for i in range(RPT // 128):
            r0 = s * RPT + i * 128
            pltpu.sync_copy(acc.at[pl.ds(r0, 128)], out_hbm.at[c, pl.ds(r0, 128)])

    return agg


_agg64 = _make_agg(64)
_agg16 = _make_agg(16)


def _dis_of(deg_ref):
    return lax.rsqrt(deg_ref[0, :N] + deg_ref[1, :N] + 1.0)[:, None]


def _b1_body(x_ref, deg_ref, w_ref, y_ref):
    xw = jnp.dot(x_ref[...], w_ref[...], preferred_element_type=jnp.float32)
    y_ref[pl.ds(0, N), :] = xw * _dis_of(deg_ref)
    y_ref[pl.ds(N, N_PAD - N), :] = jnp.zeros((N_PAD - N, y_ref.shape[1]),
                                              jnp.float32)


def _mid_body(agg_ref, y_ref, deg_ref, b_ref, w_ref, out_ref):
    dis = _dis_of(deg_ref)
    a = agg_ref[0, :N, :] + agg_ref[1, :N, :] + y_ref[:N, :]
    h = jnp.maximum(a * dis + b_ref[...][None, :], 0.0)
    out_ref[pl.ds(0, N), :] = jnp.dot(
        h, w_ref[...], preferred_element_type=jnp.float32) * dis
    out_ref[pl.ds(N, N_PAD - N), :] = jnp.zeros((N_PAD - N, out_ref.shape[1]),
                                                jnp.float32)


def _b4_body(agg_ref, y_ref, deg_ref, b_ref, out_ref):
    dis = _dis_of(deg_ref)
    t = (agg_ref[0, :N, :2] + agg_ref[1, :N, :2] + y_ref[:N, :2]) * dis
    t = t + b_ref[...][None, :]
    m = jnp.max(t, axis=1, keepdims=True)
    lse = m + jnp.log(jnp.sum(jnp.exp(t - m), axis=1, keepdims=True))
    out_ref[...] = t - lse


def _tc(body, out_rows, out_cols):
    return pl.pallas_call(
        body, out_shape=jax.ShapeDtypeStruct((out_rows, out_cols), jnp.float32))


def kernel(x, edge_index, W1, b1, W2, b2, W3, b3):
    ei = edge_index.astype(jnp.int32)
    src2 = ei[0].reshape(J_ROWS, CHUNK)
    dst2 = ei[1].reshape(J_ROWS, CHUNK)
    W3p = jnp.pad(W3, ((0, 0), (0, 16 - W3.shape[1])))

    deg = _deg_sc(dst2)
    y1 = _tc(_b1_body, N_PAD, 64)(x, deg, W1)
    a1 = _agg64(y1, src2, dst2)
    y2 = _tc(_mid_body, N_PAD, 64)(a1, y1, deg, b1, W2)
    a2 = _agg64(y2, src2, dst2)
    y3 = _tc(_mid_body, N_PAD, 16)(a2, y2, deg, b2, W3p)
    a3 = _agg16(y3, src2, dst2)
    out = _tc(_b4_body, N, 2)(a3, y3, deg, b3)
    return out


# speedup vs baseline: 39.4502x; 1.0103x over previous
import functools

import jax
import jax.numpy as jnp
from jax import lax
from jax.experimental import pallas as pl
from jax.experimental.pallas import tpu as pltpu
from jax.experimental.pallas import tpu_sc as plsc

N = 10000
N_PAD = 10240
E = 320000
CHUNK = 128
J_ROWS = E // CHUNK
NBUF = 2
TILES = 16
RPT = N_PAD // TILES

_mesh = plsc.VectorSubcoreMesh(core_axis_name="c", subcore_axis_name="s")
_sc_params = pltpu.CompilerParams(use_tc_tiling_on_sc=False)


def _row_span(wid):
    base = 78 * wid + jnp.minimum(wid, 4)
    jw = jnp.where(wid < 4, 79, 78)
    return base, jw


def _load_rows(hbm, vmem, base, jw):
    pltpu.sync_copy(hbm.at[pl.ds(base, 78)], vmem.at[pl.ds(0, 78)])

    @pl.when(jw > 78)
    def _():
        pltpu.sync_copy(hbm.at[pl.ds(base + 78, 1)], vmem.at[pl.ds(78, 1)])


@functools.partial(
    pl.kernel,
    mesh=_mesh,
    compiler_params=_sc_params,
    out_type=jax.ShapeDtypeStruct((2, N_PAD), jnp.float32),
    scratch_types=[
        pltpu.VMEM((80, CHUNK), jnp.int32),
        pltpu.VMEM((CHUNK,), jnp.float32),
        pltpu.VMEM((RPT,), jnp.float32),
        pltpu.VMEM_SHARED((N_PAD,), jnp.float32),
    ],
)
def _deg_sc(dst_hbm, out_hbm, dst_v, ones_v, zer_v, acc):
    c = lax.axis_index("c")
    s = lax.axis_index("s")
    wid = c * TILES + s
    base, jw = _row_span(wid)
    for k in range(RPT // 16):
        zer_v[pl.ds(k * 16, 16)] = jnp.zeros((16,), jnp.float32)
    for k in range(CHUNK // 16):
        ones_v[pl.ds(k * 16, 16)] = jnp.ones((16,), jnp.float32)
    pltpu.sync_copy(zer_v, acc.at[pl.ds(s * RPT, RPT)])
    _load_rows(dst_hbm, dst_v, base, jw)
    plsc.subcore_barrier()

    def body(j, carry):
        pltpu.sync_copy(ones_v, acc.at[dst_v.at[j]], add=True)
        return carry

    lax.fori_loop(0, jw, body, 0)
    plsc.subcore_barrier()
    pltpu.sync_copy(acc.at[pl.ds(s * RPT, RPT)], out_hbm.at[c, pl.ds(s * RPT, RPT)])


def _make_agg(D):
    @functools.partial(
        pl.kernel,
        mesh=_mesh,
        compiler_params=_sc_params,
        out_type=jax.ShapeDtypeStruct((2, N_PAD, D), jnp.float32),
        scratch_types=[
            pltpu.VMEM((80, CHUNK), jnp.int32),
            pltpu.VMEM((80, CHUNK), jnp.int32),
            [pltpu.VMEM((CHUNK, D), jnp.float32)] * NBUF,
            pltpu.VMEM((128, D), jnp.float32),
            pltpu.VMEM_SHARED((N_PAD, D), jnp.float32),
            pltpu.VMEM_SHARED((N_PAD, D), jnp.float32),
            [pltpu.SemaphoreType.DMA] * NBUF,
            [pltpu.SemaphoreType.DMA] * NBUF,
        ],
    )
    def agg(y_hbm, src_hbm, dst_hbm, out_hbm,
            src_v, dst_v, bufs, zer_v, acc, y_s, gs, ss):
        c = lax.axis_index("c")
        s = lax.axis_index("s")
        wid = c * TILES + s
        base, jw = _row_span(wid)
        for r in range(128):
            for k in range(D // 16):
                zer_v[r, pl.ds(k * 16, 16)] = jnp.zeros((16,), jnp.float32)
        for i in range(RPT // 128):
            pltpu.sync_copy(zer_v, acc.at[pl.ds(s * RPT + i * 128, 128)])
        pltpu.sync_copy(y_hbm.at[pl.ds(s * RPT, RPT)], y_s.at[pl.ds(s * RPT, RPT)])
        _load_rows(src_hbm, src_v, base, jw)
        _load_rows(dst_hbm, dst_v, base, jw)
        plsc.subcore_barrier()

        def fire_gather(j, b):
            pltpu.async_copy(y_s.at[src_v.at[j]], bufs[b], gs[b])

        def drain(b, sem):
            pltpu.make_async_copy(y_hbm.at[pl.ds(0, CHUNK)], bufs[b], sem).wait()

        for b in range(NBUF):
            fire_gather(b, b)

        def body(i, carry):
            j0 = NBUF * i
            for b in range(NBUF):
                j = j0 + b

                @pl.when(j < jw)
                def _():
                    drain(b, gs[b])
                    pltpu.async_copy(bufs[b], acc.at[dst_v.at[j]], ss[b], add=True)

            for b in range(NBUF):
                j = j0 + b

                @pl.when(j < jw)
                def _():
                    drain(b, ss[b])

                    @pl.when(j + NBUF < jw)
                    def _():
                        fire_gather(j + NBUF, b)

            return carry

        lax.fori_loop(0, 80 // NBUF, body, 0)
        plsc.subcore_barrier()
        for i in range(RPT // 128):
            r0 = s * RPT + i * 128
            pltpu.sync_copy(acc.at[pl.ds(r0, 128)], out_hbm.at[c, pl.ds(r0, 128)])

    return agg


_agg64 = _make_agg(64)
_agg16 = _make_agg(16)


def _dis_of(deg_ref):
    return lax.rsqrt(deg_ref[0, :N] + deg_ref[1, :N] + 1.0)[:, None]


def _b1_body(x_ref, deg_ref, w_ref, y_ref):
    xw = jnp.dot(x_ref[...], w_ref[...], preferred_element_type=jnp.float32)
    y_ref[pl.ds(0, N), :] = xw * _dis_of(deg_ref)
    y_ref[pl.ds(N, N_PAD - N), :] = jnp.zeros((N_PAD - N, y_ref.shape[1]),
                                              jnp.float32)


def _mid_body(agg_ref, y_ref, deg_ref, b_ref, w_ref, out_ref):
    dis = _dis_of(deg_ref)
    a = agg_ref[0, :N, :] + agg_ref[1, :N, :] + y_ref[:N, :]
    h = jnp.maximum(a * dis + b_ref[...][None, :], 0.0)
    out_ref[pl.ds(0, N), :] = jnp.dot(
        h, w_ref[...], preferred_element_type=jnp.float32) * dis
    out_ref[pl.ds(N, N_PAD - N), :] = jnp.zeros((N_PAD - N, out_ref.shape[1]),
                                                jnp.float32)


def _b4_body(agg_ref, y_ref, deg_ref, b_ref, out_ref):
    dis = _dis_of(deg_ref)
    t = (agg_ref[0, :N, :2] + agg_ref[1, :N, :2] + y_ref[:N, :2]) * dis
    t = t + b_ref[...][None, :]
    m = jnp.max(t, axis=1, keepdims=True)
    lse = m + jnp.log(jnp.sum(jnp.exp(t - m), axis=1, keepdims=True))
    out_ref[...] = t - lse


def _tc(body, out_rows, out_cols, n_in):
    return pl.pallas_call(
        body,
        out_shape=jax.ShapeDtypeStruct((out_rows, out_cols), jnp.float32),
        compiler_params=pltpu.CompilerParams(allow_input_fusion=[True] * n_in))


def kernel(x, edge_index, W1, b1, W2, b2, W3, b3):
    ei = edge_index.astype(jnp.int32)
    src2 = ei[0].reshape(J_ROWS, CHUNK)
    dst2 = ei[1].reshape(J_ROWS, CHUNK)
    W3p = jnp.pad(W3, ((0, 0), (0, 16 - W3.shape[1])))

    deg = _deg_sc(dst2)
    y1 = _tc(_b1_body, N_PAD, 64, 3)(x, deg, W1)
    a1 = _agg64(y1, src2, dst2)
    y2 = _tc(_mid_body, N_PAD, 64, 5)(a1, y1, deg, b1, W2)
    a2 = _agg64(y2, src2, dst2)
    y3 = _tc(_mid_body, N_PAD, 16, 5)(a2, y2, deg, b2, W3p)
    a3 = _agg16(y3, src2, dst2)
    out = _tc(_b4_body, N, 2, 4)(a3, y3, deg, b3)
    return out


# speedup vs baseline: 40.3280x; 1.0223x over previous
import functools

import jax
import jax.numpy as jnp
from jax import lax
from jax.experimental import pallas as pl
from jax.experimental.pallas import tpu as pltpu
from jax.experimental.pallas import tpu_sc as plsc

N = 10000
N_PAD = 10240
E = 320000
CHUNK = 128
J_ROWS = E // CHUNK
NBUF = 2
TILES = 16
RPT = N_PAD // TILES

_mesh = plsc.VectorSubcoreMesh(core_axis_name="c", subcore_axis_name="s")
_sc_params = pltpu.CompilerParams(use_tc_tiling_on_sc=False)


def _row_span(wid):
    base = 78 * wid + jnp.minimum(wid, 4)
    jw = jnp.where(wid < 4, 79, 78)
    return base, jw


def _load_rows(hbm, vmem, base, jw):
    pltpu.sync_copy(hbm.at[pl.ds(base, 78)], vmem.at[pl.ds(0, 78)])

    @pl.when(jw > 78)
    def _():
        pltpu.sync_copy(hbm.at[pl.ds(base + 78, 1)], vmem.at[pl.ds(78, 1)])


@functools.partial(
    pl.kernel,
    mesh=_mesh,
    compiler_params=_sc_params,
    out_type=jax.ShapeDtypeStruct((2, N_PAD), jnp.float32),
    scratch_types=[
        pltpu.VMEM((80, CHUNK), jnp.int32),
        pltpu.VMEM((CHUNK,), jnp.float32),
        pltpu.VMEM((RPT,), jnp.float32),
        pltpu.VMEM_SHARED((N_PAD,), jnp.float32),
    ],
)
def _deg_sc(dst_hbm, out_hbm, dst_v, ones_v, zer_v, acc):
    c = lax.axis_index("c")
    s = lax.axis_index("s")
    wid = c * TILES + s
    base, jw = _row_span(wid)
    for k in range(RPT // 16):
        zer_v[pl.ds(k * 16, 16)] = jnp.zeros((16,), jnp.float32)
    for k in range(CHUNK // 16):
        ones_v[pl.ds(k * 16, 16)] = jnp.ones((16,), jnp.float32)
    pltpu.sync_copy(zer_v, acc.at[pl.ds(s * RPT, RPT)])
    _load_rows(dst_hbm, dst_v, base, jw)
    plsc.subcore_barrier()

    def body(j, carry):
        pltpu.sync_copy(ones_v, acc.at[dst_v.at[j]], add=True)
        return carry

    lax.fori_loop(0, jw, body, 0)
    plsc.subcore_barrier()
    pltpu.sync_copy(acc.at[pl.ds(s * RPT, RPT)], out_hbm.at[c, pl.ds(s * RPT, RPT)])


def _make_agg(D):
    @functools.partial(
        pl.kernel,
        mesh=_mesh,
        compiler_params=_sc_params,
        out_type=jax.ShapeDtypeStruct((2, N_PAD, D), jnp.float32),
        scratch_types=[
            pltpu.VMEM((80, CHUNK), jnp.int32),
            pltpu.VMEM((80, CHUNK), jnp.int32),
            [pltpu.VMEM((CHUNK, D), jnp.float32)] * NBUF,
            pltpu.VMEM((128, D), jnp.float32),
            pltpu.VMEM_SHARED((N_PAD, D), jnp.float32),
            pltpu.VMEM_SHARED((N_PAD, D), jnp.float32),
            [pltpu.SemaphoreType.DMA] * NBUF,
            [pltpu.SemaphoreType.DMA] * NBUF,
        ],
    )
    def agg(y_hbm, src_hbm, dst_hbm, out_hbm,
            src_v, dst_v, bufs, zer_v, acc, y_s, gs, ss):
        c = lax.axis_index("c")
        s = lax.axis_index("s")
        wid = c * TILES + s
        base, jw = _row_span(wid)
        for r in range(128):
            for k in range(D // 16):
                zer_v[r, pl.ds(k * 16, 16)] = jnp.zeros((16,), jnp.float32)
        stage = pltpu.async_copy(y_hbm.at[pl.ds(s * RPT, RPT)],
                                 y_s.at[pl.ds(s * RPT, RPT)], gs[0])
        for i in range(RPT // 128):
            pltpu.sync_copy(zer_v, acc.at[pl.ds(s * RPT + i * 128, 128)])
        _load_rows(src_hbm, src_v, base, jw)
        _load_rows(dst_hbm, dst_v, base, jw)
        stage.wait()
        plsc.subcore_barrier()

        def fire_gather(j, b):
            pltpu.async_copy(y_s.at[src_v.at[j]], bufs[b], gs[b])

        def drain(b, sem):
            pltpu.make_async_copy(y_hbm.at[pl.ds(0, CHUNK)], bufs[b], sem).wait()

        for b in range(NBUF):
            fire_gather(b, b)

        def body(i, carry):
            j0 = NBUF * i
            for b in range(NBUF):
                j = j0 + b

                @pl.when(j < jw)
                def _():
                    drain(b, gs[b])
                    pltpu.async_copy(bufs[b], acc.at[dst_v.at[j]], ss[b], add=True)

            for b in range(NBUF):
                j = j0 + b

                @pl.when(j < jw)
                def _():
                    drain(b, ss[b])

                    @pl.when(j + NBUF < jw)
                    def _():
                        fire_gather(j + NBUF, b)

            return carry

        lax.fori_loop(0, 80 // NBUF, body, 0)
        plsc.subcore_barrier()
        for i in range(RPT // 128):
            r0 = s * RPT + i * 128
            pltpu.sync_copy(acc.at[pl.ds(r0, 128)], out_hbm.at[c, pl.ds(r0, 128)])

    return agg


_agg64 = _make_agg(64)
_agg16 = _make_agg(16)


def _dis_of(deg_ref):
    return lax.rsqrt(deg_ref[0, :N] + deg_ref[1, :N] + 1.0)[:, None]


def _b1_body(x_ref, deg_ref, w_ref, y_ref):
    xw = jnp.dot(x_ref[...], w_ref[...], preferred_element_type=jnp.float32)
    y_ref[pl.ds(0, N), :] = xw * _dis_of(deg_ref)
    y_ref[pl.ds(N, N_PAD - N), :] = jnp.zeros((N_PAD - N, y_ref.shape[1]),
                                              jnp.float32)


def _mid_body(agg_ref, y_ref, deg_ref, b_ref, w_ref, out_ref):
    dis = _dis_of(deg_ref)
    a = agg_ref[0, :N, :] + agg_ref[1, :N, :] + y_ref[:N, :]
    h = jnp.maximum(a * dis + b_ref[...][None, :], 0.0)
    out_ref[pl.ds(0, N), :] = jnp.dot(
        h, w_ref[...], preferred_element_type=jnp.float32) * dis
    out_ref[pl.ds(N, N_PAD - N), :] = jnp.zeros((N_PAD - N, out_ref.shape[1]),
                                                jnp.float32)


def _b4_body(agg_ref, y_ref, deg_ref, b_ref, out_ref):
    dis = _dis_of(deg_ref)
    t = (agg_ref[0, :N, :2] + agg_ref[1, :N, :2] + y_ref[:N, :2]) * dis
    t = t + b_ref[...][None, :]
    m = jnp.max(t, axis=1, keepdims=True)
    lse = m + jnp.log(jnp.sum(jnp.exp(t - m), axis=1, keepdims=True))
    out_ref[...] = t - lse


def _tc(body, out_rows, out_cols, n_in):
    return pl.pallas_call(
        body,
        out_shape=jax.ShapeDtypeStruct((out_rows, out_cols), jnp.float32),
        compiler_params=pltpu.CompilerParams(allow_input_fusion=[True] * n_in))


def kernel(x, edge_index, W1, b1, W2, b2, W3, b3):
    ei = edge_index.astype(jnp.int32)
    src2 = ei[0].reshape(J_ROWS, CHUNK)
    dst2 = ei[1].reshape(J_ROWS, CHUNK)
    W3p = jnp.pad(W3, ((0, 0), (0, 16 - W3.shape[1])))

    deg = _deg_sc(dst2)
    y1 = _tc(_b1_body, N_PAD, 64, 3)(x, deg, W1)
    a1 = _agg64(y1, src2, dst2)
    y2 = _tc(_mid_body, N_PAD, 64, 5)(a1, y1, deg, b1, W2)
    a2 = _agg64(y2, src2, dst2)
    y3 = _tc(_mid_body, N_PAD, 16, 5)(a2, y2, deg, b2, W3p)
    a3 = _agg16(y3, src2, dst2)
    out = _tc(_b4_body, N, 2, 4)(a3, y3, deg, b3)
    return out
